# Initial kernel scaffold; baseline (speedup 1.0000x reference)
import functools

import jax
import jax.numpy as jnp
from jax import lax
from jax.experimental import pallas as pl
from jax.experimental.pallas import tpu as pltpu
from jax.experimental.pallas import tpu_sc as plsc

_LANES = 16
_C = 128
_K = 8
_BLK = _C * _K


@functools.partial(jax.jit, static_argnums=(2, 3))
def _sc_gather(idx, table, total_rows, v_rows):
    info = plsc.get_sparse_core_info()
    nw = info.num_cores * info.num_subcores
    nc = info.num_cores
    half = table.shape[1]
    per_w = total_rows // nw
    n_blk = per_w // _BLK
    assert per_w % _BLK == 0

    mesh = plsc.VectorSubcoreMesh(core_axis_name="c", subcore_axis_name="s")

    @functools.partial(
        pl.kernel,
        mesh=mesh,
        out_type=jax.ShapeDtypeStruct((total_rows, half), jnp.float32),
        scratch_types=[
            pltpu.VMEM((_BLK,), jnp.int32),
            pltpu.VMEM((_BLK, half), jnp.float32),
            pltpu.SemaphoreType.DMA,
        ],
    )
    def k(idx_hbm, table_hbm, out_hbm, idx_v, rows_v, sem):
        wid = lax.axis_index("s") * nc + lax.axis_index("c")
        base = wid * per_w
        offs = (lax.iota(jnp.int32, _LANES) % 2) * jnp.int32(v_rows)

        def block(i, _):
            row0 = base + i * _BLK
            pltpu.sync_copy(idx_hbm.at[pl.ds(row0, _BLK)], idx_v)

            def adj(v, _):
                sl = pl.ds(v * _LANES, _LANES)
                idx_v[sl] = idx_v[sl] + offs
                return 0

            lax.fori_loop(0, _BLK // _LANES, adj, 0)

            copies = [
                pltpu.make_async_copy(
                    table_hbm.at[idx_v.at[pl.ds(j * _C, _C)]],
                    rows_v.at[pl.ds(j * _C, _C)],
                    sem,
                )
                for j in range(_K)
            ]
            for c in copies:
                c.start()
            for c in copies:
                c.wait()
            pltpu.sync_copy(rows_v, out_hbm.at[pl.ds(row0, _BLK)])
            return 0

        lax.fori_loop(0, n_blk, block, 0)

    return k(idx, table)


def kernel(spans, start_emb, end_emb):
    b, n, _ = spans.shape
    v_rows, half = start_emb.shape
    table = jnp.concatenate([start_emb, end_emb], axis=0)
    idx = spans.reshape(-1)
    out = _sc_gather(idx, table, b * n * 2, v_rows)
    return out.reshape(b, n, 2 * half)


# speedup vs baseline: 4.3074x; 4.3074x over previous
import functools

import jax
import jax.numpy as jnp
from jax import lax
from jax.experimental import pallas as pl
from jax.experimental.pallas import tpu as pltpu
from jax.experimental.pallas import tpu_sc as plsc

_LANES = 16
_C = 128
_K = 8
_BLK = _C * _K


@functools.partial(jax.jit, static_argnums=(2, 3))
def _sc_gather(idx, table, total_rows, v_rows):
    info = plsc.get_sparse_core_info()
    nw = info.num_cores * info.num_subcores
    nc = info.num_cores
    half = table.shape[1]
    per_w = total_rows // nw
    n_blk = per_w // _BLK
    assert per_w % _BLK == 0

    mesh = plsc.VectorSubcoreMesh(core_axis_name="c", subcore_axis_name="s")

    @functools.partial(
        pl.kernel,
        mesh=mesh,
        out_type=jax.ShapeDtypeStruct((total_rows, half), jnp.float32),
        scratch_types=[
            pltpu.VMEM((_BLK,), jnp.int32),
            pltpu.VMEM((_BLK, half), jnp.float32),
            pltpu.SemaphoreType.DMA,
        ],
        compiler_params=pltpu.CompilerParams(use_tc_tiling_on_sc=False),
    )
    def k(idx_hbm, table_hbm, out_hbm, idx_v, rows_v, sem):
        wid = lax.axis_index("s") * nc + lax.axis_index("c")
        base = wid * per_w
        offs = (lax.iota(jnp.int32, _LANES) % 2) * jnp.int32(v_rows)

        def block(i, _):
            row0 = base + i * _BLK
            pltpu.sync_copy(idx_hbm.at[pl.ds(row0, _BLK)], idx_v)

            def adj(v, _):
                sl = pl.ds(v * _LANES, _LANES)
                idx_v[sl] = idx_v[sl] + offs
                return 0

            lax.fori_loop(0, _BLK // _LANES, adj, 0)

            copies = [
                pltpu.make_async_copy(
                    table_hbm.at[idx_v.at[pl.ds(j * _C, _C)]],
                    rows_v.at[pl.ds(j * _C, _C)],
                    sem,
                )
                for j in range(_K)
            ]
            for c in copies:
                c.start()
            for c in copies:
                c.wait()
            pltpu.sync_copy(rows_v, out_hbm.at[pl.ds(row0, _BLK)])
            return 0

        lax.fori_loop(0, n_blk, block, 0)

    return k(idx, table)


def kernel(spans, start_emb, end_emb):
    b, n, _ = spans.shape
    v_rows, half = start_emb.shape
    table = jnp.concatenate([start_emb, end_emb], axis=0)
    idx = spans.reshape(-1)
    out = _sc_gather(idx, table, b * n * 2, v_rows)
    return out.reshape(b, n, 2 * half)


# speedup vs baseline: 18.0134x; 4.1820x over previous
import functools

import jax
import jax.numpy as jnp
from jax import lax
from jax.experimental import pallas as pl
from jax.experimental.pallas import tpu as pltpu
from jax.experimental.pallas import tpu_sc as plsc

_LANES = 16
_C = 128


@functools.partial(jax.jit, static_argnums=(2,))
def _sc_gather(idx3, table, v_rows):
    info = plsc.get_sparse_core_info()
    nw = info.num_cores * info.num_subcores
    nc = info.num_cores
    half = table.shape[1]
    n_seq, two_nb, lanes = idx3.shape
    nb = two_nb // 2
    f_rows = 2 * n_seq
    per_w = (nb // nw) * lanes * f_rows
    total_rows = nb * lanes * f_rows
    assert nb % nw == 0 and lanes == _C and f_rows % 8 == 0

    hb = 2 * _C
    n_blk = per_w // hb
    n_slots = 3
    n_full = hb // _C
    tail = hb - n_full * _C
    assert per_w % hb == 0 and tail % 8 == 0

    mesh = plsc.VectorSubcoreMesh(core_axis_name="c", subcore_axis_name="s")

    @functools.partial(
        pl.kernel,
        mesh=mesh,
        out_type=jax.ShapeDtypeStruct((total_rows, half), jnp.float32),
        scratch_types=[
            pltpu.VMEM((n_seq, 2, _C), jnp.int32),
            pltpu.VMEM((n_slots, hb), jnp.int32),
            pltpu.VMEM((n_slots, hb, half), jnp.float32),
            [pltpu.SemaphoreType.DMA] * n_slots,
            [pltpu.SemaphoreType.DMA] * n_slots,
        ],
        compiler_params=pltpu.CompilerParams(
            use_tc_tiling_on_sc=False, needs_layout_passes=False),
    )
    def k(idx_hbm, table_hbm, out_hbm, idx_all, idx_perm, rows_v, gsem, ssem):
        wid = lax.axis_index("s") * nc + lax.axis_index("c")
        base = wid * per_w
        iota = lax.iota(jnp.int32, _LANES)
        p_vec = iota & 1
        offs = p_vec * jnp.int32(v_rows)

        pltpu.sync_copy(idx_hbm.at[:, pl.ds(wid * 2, 2), :], idx_all)

        def build(blk, slot):
            r0 = blk * hb

            def step(v, _):
                r_vec = r0 + v * _LANES + iota
                bl_vec = r_vec // jnp.int32(f_rows)
                n_vec = (r_vec - bl_vec * jnp.int32(f_rows)) >> 1
                vals = plsc.load_gather(idx_all, [n_vec, p_vec, bl_vec])
                idx_perm[slot, pl.ds(v * _LANES, _LANES)] = vals + offs
                return 0

            lax.fori_loop(0, hb // _LANES, step, 0)

        def descriptors(slot):
            cps = []
            for j in range(n_full):
                cps.append(pltpu.make_async_copy(
                    table_hbm.at[idx_perm.at[slot, pl.ds(j * _C, _C)]],
                    rows_v.at[slot, pl.ds(j * _C, _C)],
                    gsem[slot],
                ))
            if tail:
                cps.append(pltpu.make_async_copy(
                    table_hbm.at[idx_perm.at[slot, pl.ds(n_full * _C, tail)]],
                    rows_v.at[slot, pl.ds(n_full * _C, tail)],
                    gsem[slot],
                ))
            return cps

        def fire(slot):
            for c in descriptors(slot):
                c.start()

        def wait_gathers(slot):
            for c in descriptors(slot):
                c.wait()

        def scatter_desc(slot, blk):
            return pltpu.make_async_copy(
                rows_v.at[slot],
                out_hbm.at[pl.ds(base + blk * hb, hb)],
                ssem[slot],
            )

        for s in range(n_slots):
            build(s, s)
            fire(s)

        def blk_loop(it, _):
            for s in range(n_slots):
                blk = it * n_slots + s

                @pl.when(blk < n_blk)
                def _():
                    wait_gathers(s)
                    scatter_desc(s, blk).start()

                    @pl.when(blk + n_slots < n_blk)
                    def _():
                        build(blk + n_slots, s)

                    scatter_desc(s, blk).wait()

                    @pl.when(blk + n_slots < n_blk)
                    def _():
                        fire(s)

            return 0

        lax.fori_loop(0, (n_blk + n_slots - 1) // n_slots, blk_loop, 0)

    return k(idx3, table)


def kernel(spans, start_emb, end_emb):
    b, n, _ = spans.shape
    v_rows, half = start_emb.shape
    table = jnp.concatenate([start_emb, end_emb], axis=0)
    idx3 = spans.reshape(b // 128, 128, n, 2).transpose(2, 0, 3, 1)
    idx3 = idx3.reshape(n, (b // 128) * 2, 128)
    out = _sc_gather(idx3, table, v_rows)
    return out.reshape(b, n, 2 * half)
